# baseline (speedup 1.0000x reference)
import functools

import jax
import jax.numpy as jnp
from jax import lax
from jax.experimental import pallas as pl
from jax.experimental.pallas import tpu as pltpu
from jax.experimental.pallas import tpu_sc as plsc

_NUM_WORKERS = 32
_LANES = 16


def kernel(x, row_embed, col_embed):
    b, _, h, w = x.shape
    n_rows, d = col_embed.shape
    c_total = 2 * d

    mesh = plsc.VectorSubcoreMesh(core_axis_name="c", subcore_axis_name="s")

    @functools.partial(
        pl.kernel,
        mesh=mesh,
        out_type=jax.ShapeDtypeStruct((b, h, w, c_total), jnp.float32),
        scratch_types=[
            pltpu.VMEM((d,), jnp.float32),
            pltpu.VMEM((w, c_total), jnp.float32),
            pltpu.SemaphoreType.DMA,
        ],
        compiler_params=pltpu.CompilerParams(needs_layout_passes=False),
    )
    def pos_kernel(row_hbm, col_hbm, out_hbm, row_v, blk, sem):
        wid = lax.axis_index("s") * 2 + lax.axis_index("c")
        hh = wid

        pltpu.sync_copy(col_hbm.at[pl.ds(0, w), :], blk.at[:, pl.ds(0, d)])
        pltpu.sync_copy(row_hbm.at[hh], row_v)

        def w_body(ww, carry):
            def k_body(k, c2):
                blk[ww, pl.ds(d + k * _LANES, _LANES)] = row_v[
                    pl.ds(k * _LANES, _LANES)
                ]
                return c2

            return lax.fori_loop(0, d // _LANES, k_body, carry)

        lax.fori_loop(0, w, w_body, 0)

        def fire(bb, carry):
            pltpu.async_copy(blk, out_hbm.at[bb, hh], sem)
            return carry

        lax.fori_loop(0, b, fire, 0)

        def drain(bb, carry):
            pltpu.make_async_copy(blk, out_hbm.at[bb, hh], sem).wait()
            return carry

        lax.fori_loop(0, b, drain, 0)

    out = pos_kernel(row_embed, col_embed)
    return jnp.transpose(out, (0, 3, 1, 2))


# speedup vs baseline: 1.0053x; 1.0053x over previous
import functools

import jax
import jax.numpy as jnp
from jax import lax
from jax.experimental import pallas as pl
from jax.experimental.pallas import tpu as pltpu
from jax.experimental.pallas import tpu_sc as plsc

_NUM_WORKERS = 32
_LANES = 16


def kernel(x, row_embed, col_embed):
    b, _, h, w = x.shape
    n_rows, d = col_embed.shape
    c_total = 2 * d

    mesh = plsc.VectorSubcoreMesh(core_axis_name="c", subcore_axis_name="s")

    @functools.partial(
        pl.kernel,
        mesh=mesh,
        out_type=jax.ShapeDtypeStruct((b, h, w, c_total), jnp.float32),
        scratch_types=[
            pltpu.VMEM((d,), jnp.float32),
            pltpu.VMEM((w, c_total), jnp.float32),
            pltpu.SemaphoreType.DMA,
        ],
        compiler_params=pltpu.CompilerParams(needs_layout_passes=False),
    )
    def pos_kernel(row_hbm, col_hbm, out_hbm, row_v, blk, sem):
        wid = lax.axis_index("s") * 2 + lax.axis_index("c")
        hh = wid

        pltpu.sync_copy(col_hbm.at[pl.ds(0, w), :], blk.at[:, pl.ds(0, d)])
        pltpu.sync_copy(row_hbm.at[hh], row_v)

        def w_body(ww, carry):
            def k_body(k, c2):
                blk[ww, pl.ds(d + k * _LANES, _LANES)] = row_v[
                    pl.ds(k * _LANES, _LANES)
                ]
                return c2

            return lax.fori_loop(0, d // _LANES, k_body, carry)

        lax.fori_loop(0, w, w_body, 0)

        copies = [
            pltpu.async_copy(blk, out_hbm.at[bb, hh], sem) for bb in range(b)
        ]
        for cp in copies:
            cp.wait()

    out = pos_kernel(row_embed, col_embed)
    return jnp.transpose(out, (0, 3, 1, 2))


# speedup vs baseline: 1.0078x; 1.0024x over previous
import functools

import jax
import jax.numpy as jnp
from jax import lax
from jax.experimental import pallas as pl
from jax.experimental.pallas import tpu as pltpu
from jax.experimental.pallas import tpu_sc as plsc

_NUM_WORKERS = 32
_LANES = 16


def kernel(x, row_embed, col_embed):
    b, _, h, w = x.shape
    n_rows, d = col_embed.shape
    c_total = 2 * d

    mesh = plsc.VectorSubcoreMesh(core_axis_name="c", subcore_axis_name="s")

    @functools.partial(
        pl.kernel,
        mesh=mesh,
        out_type=jax.ShapeDtypeStruct((b, h, w, c_total), jnp.float32),
        scratch_types=[
            pltpu.VMEM((d,), jnp.float32),
            pltpu.VMEM((w, c_total), jnp.float32),
            pltpu.SemaphoreType.DMA,
        ],
        compiler_params=pltpu.CompilerParams(needs_layout_passes=False),
    )
    def pos_kernel(row_hbm, col_hbm, out_hbm, row_v, blk, sem):
        wid = lax.axis_index("s") * 2 + lax.axis_index("c")
        hh = wid

        col_cp = pltpu.async_copy(
            col_hbm.at[pl.ds(0, w), :], blk.at[:, pl.ds(0, d)], sem
        )
        pltpu.sync_copy(row_hbm.at[hh], row_v)

        segs = [row_v[pl.ds(k * _LANES, _LANES)] for k in range(d // _LANES)]

        def w_body(ww, carry):
            for k, v in enumerate(segs):
                blk[ww, pl.ds(d + k * _LANES, _LANES)] = v
            return carry

        lax.fori_loop(0, w, w_body, 0)
        col_cp.wait()

        copies = [
            pltpu.async_copy(blk, out_hbm.at[bb, hh], sem) for bb in range(b)
        ]
        for cp in copies:
            cp.wait()

    out = pos_kernel(row_embed, col_embed)
    return jnp.transpose(out, (0, 3, 1, 2))


# speedup vs baseline: 1.0591x; 1.0509x over previous
import functools

import jax
import jax.numpy as jnp
from jax import lax
from jax.experimental import pallas as pl
from jax.experimental.pallas import tpu as pltpu
from jax.experimental.pallas import tpu_sc as plsc

_NUM_WORKERS = 32
_LANES = 16


def kernel(x, row_embed, col_embed):
    b, _, h, w = x.shape
    n_rows, d = col_embed.shape
    c_total = 2 * d

    mesh = plsc.VectorSubcoreMesh(core_axis_name="c", subcore_axis_name="s")

    @functools.partial(
        pl.kernel,
        mesh=mesh,
        out_type=jax.ShapeDtypeStruct((b, h, w, c_total), jnp.float32),
        scratch_types=[
            pltpu.VMEM((d,), jnp.float32),
            pltpu.VMEM((w, c_total), jnp.float32),
            pltpu.SemaphoreType.DMA,
        ],
        compiler_params=pltpu.CompilerParams(needs_layout_passes=False),
    )
    def pos_kernel(row_hbm, col_hbm, out_hbm, row_v, blk, sem):
        wid = lax.axis_index("s") * 2 + lax.axis_index("c")
        hh = wid

        pltpu.sync_copy(col_hbm.at[pl.ds(0, w), :], blk.at[:, pl.ds(0, d)])
        pltpu.sync_copy(row_hbm.at[hh], row_v)

        segs = [row_v[pl.ds(k * _LANES, _LANES)] for k in range(d // _LANES)]

        def w_body(ww, carry):
            for k, v in enumerate(segs):
                blk[ww, pl.ds(d + k * _LANES, _LANES)] = v
            return carry

        lax.fori_loop(0, w, w_body, 0)

        copies = [
            pltpu.async_copy(blk, out_hbm.at[bb, hh], sem) for bb in range(b)
        ]
        for cp in copies:
            cp.wait()

    out = pos_kernel(row_embed, col_embed)
    return jnp.transpose(out, (0, 3, 1, 2))
